# Initial kernel scaffold; baseline (speedup 1.0000x reference)
import jax
import jax.numpy as jnp
from jax.experimental import pallas as pl


def kernel(x, angle):
    raise NotImplementedError("write your pallas kernel here")


# speedup vs baseline: 23.3108x; 23.3108x over previous
import functools

import jax
import jax.numpy as jnp
from jax import lax
from jax.experimental import pallas as pl
from jax.experimental.pallas import tpu as pltpu
from jax.experimental.pallas import tpu_sc as plsc

_D = 4096
_B = 32
_NC = 2
_NS = 16
_NW = _NC * _NS
_L = 16
_PER = _D * _B // _NW
_ROWS_PER_W = _D // _NW
_STEPS = _PER // _L

_mesh = plsc.VectorSubcoreMesh(core_axis_name="c", subcore_axis_name="s")


@functools.partial(
    pl.kernel,
    out_type=(
        jax.ShapeDtypeStruct((_D * _B,), jnp.float32),
        jax.ShapeDtypeStruct((_D * _B,), jnp.float32),
    ),
    mesh=_mesh,
    scratch_types=(
        pltpu.VMEM((_PER,), jnp.float32),
        pltpu.VMEM((2 * _L,), jnp.float32),
        pltpu.VMEM((_PER,), jnp.float32),
        pltpu.VMEM((_PER,), jnp.float32),
    ),
)
def _crz_sc(x_hbm, cs_hbm, re_hbm, im_hbm, xv, csv, rev, imv):
    cid = lax.axis_index("c")
    sid = lax.axis_index("s")
    wid = sid * _NC + cid
    loc = (wid >> 4) & 1
    kdig = (wid >> 3) & 1
    base = wid * _PER
    pltpu.sync_copy(x_hbm.at[pl.ds(base, _PER)], xv)
    pltpu.sync_copy(cs_hbm, csv)
    vc = csv[pl.ds(0, _L)]
    vs = csv[pl.ds(_L, _L)]
    locf = loc.astype(jnp.float32)
    sgn = (2 * kdig - 1).astype(jnp.float32)
    vcr = 1.0 + locf * (vc - 1.0)
    vci = (locf * sgn) * vs

    def step(j, carry):
        off = j * _L
        v = xv[pl.ds(off, _L)]
        rev[pl.ds(off, _L)] = vcr * v
        imv[pl.ds(off, _L)] = vci * v
        return carry

    lax.fori_loop(0, _STEPS, step, 0)
    pltpu.sync_copy(rev, re_hbm.at[pl.ds(base, _PER)])
    pltpu.sync_copy(imv, im_hbm.at[pl.ds(base, _PER)])


def kernel(x, angle):
    half = angle[0] * jnp.float32(0.5)
    cs = jnp.concatenate(
        [jnp.full((_L,), jnp.cos(half), jnp.float32),
         jnp.full((_L,), jnp.sin(half), jnp.float32)]
    )
    re, im = _crz_sc(x.reshape(-1), cs)
    return lax.complex(re.reshape(_D, _B), im.reshape(_D, _B))
